# Initial kernel scaffold; baseline (speedup 1.0000x reference)
import jax
import jax.numpy as jnp
from jax.experimental import pallas as pl

_N = 10000
_H = 128
_G = 256
_BM = 400


def _mm_kernel(a_ref, w_ref, o_ref):
    o_ref[...] = jnp.dot(a_ref[...], w_ref[...],
                         preferred_element_type=jnp.float32)


def _hop1_kernel(adj_ref, y_ref, b_ref, w_ref, o_ref):
    t = jnp.dot(adj_ref[...], y_ref[...], preferred_element_type=jnp.float32)
    t = jnp.maximum(t + b_ref[...], 0.0)
    o_ref[...] = jnp.dot(t, w_ref[...], preferred_element_type=jnp.float32)


def _hop2_kernel(adj_ref, y_ref, b_ref, o_ref):
    t = jnp.dot(adj_ref[...], y_ref[...], preferred_element_type=jnp.float32)
    o_ref[...] = jnp.maximum(t + b_ref[...], 0.0)


def _finalize_kernel(x_ref, idxf_ref, bn1g_ref, bn1b_ref, bn2g_ref, bn2b_ref,
                     fc1w_ref, fc1b_ref, fc2w_ref, fc2b_ref, o_ref):
    x = x_ref[...]
    n = jnp.float32(_N)
    mean = jnp.sum(x, axis=0, keepdims=True) / n
    var = jnp.sum(x * x, axis=0, keepdims=True) / n - mean * mean
    a = bn1g_ref[...] * jax.lax.rsqrt(var + 1e-5)
    c = bn1b_ref[...] - a * mean
    xn = a * x + c
    seg_ids = jax.lax.broadcasted_iota(jnp.float32, (_G, _N), 0)
    idx = jnp.broadcast_to(idxf_ref[...], (_G, _N))
    s = jnp.where(seg_ids == idx, 1.0, 0.0)
    pooled = jnp.dot(s, xn, preferred_element_type=jnp.float32)
    g = jnp.float32(_G)
    mean2 = jnp.sum(pooled, axis=0, keepdims=True) / g
    var2 = jnp.sum(pooled * pooled, axis=0, keepdims=True) / g - mean2 * mean2
    p = bn2g_ref[...] * (pooled - mean2) * jax.lax.rsqrt(var2 + 1e-5) \
        + bn2b_ref[...]
    h = jnp.maximum(jnp.dot(p, fc1w_ref[...],
                            preferred_element_type=jnp.float32)
                    + fc1b_ref[...], 0.0)
    o = jnp.dot(h, fc2w_ref[...], preferred_element_type=jnp.float32) \
        + fc2b_ref[...]
    m = jnp.max(o, axis=1, keepdims=True)
    e = jnp.exp(o - m)
    lse = jnp.log(jnp.sum(e, axis=1, keepdims=True)) + m
    o_ref[...] = o - lse


def _hop(adj, y, b, w=None):
    grid = (_N // _BM,)
    in_specs = [
        pl.BlockSpec((_BM, _N), lambda i: (i, 0)),
        pl.BlockSpec((_N, _H), lambda i: (0, 0)),
        pl.BlockSpec((1, _H), lambda i: (0, 0)),
    ]
    args = [adj, y, b]
    if w is not None:
        in_specs.append(pl.BlockSpec((_H, _H), lambda i: (0, 0)))
        args.append(w)
        body = _hop1_kernel
    else:
        body = _hop2_kernel
    return pl.pallas_call(
        body,
        grid=grid,
        in_specs=in_specs,
        out_specs=pl.BlockSpec((_BM, _H), lambda i: (i, 0)),
        out_shape=jax.ShapeDtypeStruct((_N, _H), jnp.float32),
    )(*args)


def kernel(adj, features, idx, W0, b0, W1, b1, bn1_g, bn1_b, bn2_g, bn2_b,
           fc1_W, fc1_b, fc2_W, fc2_b):
    row = lambda v: v.reshape(1, -1).astype(jnp.float32)
    y0 = pl.pallas_call(
        _mm_kernel,
        out_shape=jax.ShapeDtypeStruct((_N, _H), jnp.float32),
    )(features, W0)
    y1 = _hop(adj, y0, row(b0), W1)
    x2 = _hop(adj, y1, row(b1))
    idxf = idx.astype(jnp.float32).reshape(1, _N)
    out = pl.pallas_call(
        _finalize_kernel,
        out_shape=jax.ShapeDtypeStruct((_G, 16), jnp.float32),
    )(x2, idxf, row(bn1_g), row(bn1_b), row(bn2_g), row(bn2_b),
      fc1_W, row(fc1_b), fc2_W, row(fc2_b))
    return out


# speedup vs baseline: 1.3472x; 1.3472x over previous
import jax
import jax.numpy as jnp
from jax.experimental import pallas as pl

_N = 10000
_H = 128
_G = 256
_BM = 400


def _mm_kernel(a_ref, w_ref, o_ref):
    o_ref[...] = jnp.dot(a_ref[...], w_ref[...],
                         preferred_element_type=jnp.float32)


def _hop1_kernel(adj_ref, y_ref, b_ref, w_ref, o_ref):
    t = jnp.dot(adj_ref[...], y_ref[...], preferred_element_type=jnp.float32)
    t = jnp.maximum(t + b_ref[...], 0.0)
    o_ref[...] = jnp.dot(t, w_ref[...], preferred_element_type=jnp.float32)


def _hop2_kernel(adj_ref, y_ref, b_ref, o_ref):
    t = jnp.dot(adj_ref[...], y_ref[...], preferred_element_type=jnp.float32)
    o_ref[...] = jnp.maximum(t + b_ref[...], 0.0)


def _finalize_kernel(x_ref, idxf_ref, bn1g_ref, bn1b_ref, bn2g_ref, bn2b_ref,
                     fc1w_ref, fc1b_ref, fc2w_ref, fc2b_ref, o_ref):
    x = x_ref[...]
    n = jnp.float32(_N)
    mean = jnp.sum(x, axis=0, keepdims=True) / n
    var = jnp.sum(x * x, axis=0, keepdims=True) / n - mean * mean
    a = bn1g_ref[...] * jax.lax.rsqrt(var + 1e-5)
    c = bn1b_ref[...] - a * mean
    xn = a * x + c
    seg_ids = jax.lax.broadcasted_iota(jnp.int32, (_G, _N), 0).astype(
        jnp.float32)
    idx = jnp.broadcast_to(idxf_ref[...], (_G, _N))
    s = jnp.where(seg_ids == idx, 1.0, 0.0)
    pooled = jnp.dot(s, xn, preferred_element_type=jnp.float32)
    g = jnp.float32(_G)
    mean2 = jnp.sum(pooled, axis=0, keepdims=True) / g
    var2 = jnp.sum(pooled * pooled, axis=0, keepdims=True) / g - mean2 * mean2
    p = bn2g_ref[...] * (pooled - mean2) * jax.lax.rsqrt(var2 + 1e-5) \
        + bn2b_ref[...]
    h = jnp.maximum(jnp.dot(p, fc1w_ref[...],
                            preferred_element_type=jnp.float32)
                    + fc1b_ref[...], 0.0)
    o = jnp.dot(h, fc2w_ref[...], preferred_element_type=jnp.float32) \
        + fc2b_ref[...]
    m = jnp.max(o, axis=1, keepdims=True)
    e = jnp.exp(o - m)
    lse = jnp.log(jnp.sum(e, axis=1, keepdims=True)) + m
    o_ref[...] = o - lse


def _hop(adj, y, b, w=None):
    grid = (_N // _BM,)
    in_specs = [
        pl.BlockSpec((_BM, _N), lambda i: (i, 0)),
        pl.BlockSpec((_N, _H), lambda i: (0, 0)),
        pl.BlockSpec((1, _H), lambda i: (0, 0)),
    ]
    args = [adj, y, b]
    if w is not None:
        in_specs.append(pl.BlockSpec((_H, _H), lambda i: (0, 0)))
        args.append(w)
        body = _hop1_kernel
    else:
        body = _hop2_kernel
    return pl.pallas_call(
        body,
        grid=grid,
        in_specs=in_specs,
        out_specs=pl.BlockSpec((_BM, _H), lambda i: (i, 0)),
        out_shape=jax.ShapeDtypeStruct((_N, _H), jnp.float32),
    )(*args)


def kernel(adj, features, idx, W0, b0, W1, b1, bn1_g, bn1_b, bn2_g, bn2_b,
           fc1_W, fc1_b, fc2_W, fc2_b):
    row = lambda v: v.reshape(1, -1).astype(jnp.float32)
    y0 = pl.pallas_call(
        _mm_kernel,
        out_shape=jax.ShapeDtypeStruct((_N, _H), jnp.float32),
    )(features, W0)
    y1 = _hop(adj, y0, row(b0), W1)
    x2 = _hop(adj, y1, row(b1))
    idxf = idx.astype(jnp.float32).reshape(1, _N)
    out = pl.pallas_call(
        _finalize_kernel,
        out_shape=jax.ShapeDtypeStruct((_G, 16), jnp.float32),
    )(x2, idxf, row(bn1_g), row(bn1_b), row(bn2_g), row(bn2_b),
      fc1_W, row(fc1_b), fc2_W, row(fc2_b))
    return out
